# baseline (speedup 1.0000x reference)
import functools

import jax
import jax.numpy as jnp
from jax import lax
from jax.experimental import pallas as pl
from jax.experimental.pallas import tpu as pltpu
from jax.experimental.pallas import tpu_sc as plsc

_G = 64
_N = 10000
_E = 320000
_D = 128
_NPAD = 10240
_C = 128
_EPT = _E // 32
_NCHUNK = (_EPT + _C - 1) // _C
_RPT = _NPAD // 16
_HROWS = 128
_NPT = _NPAD // 32


def _mesh():
    return plsc.VectorSubcoreMesh(core_axis_name="c", subcore_axis_name="s")


def _edgepass_body(vals, rowh, colh, out, colv, rowv, rows, acc, sem):
    d = rows.shape[1]
    c = lax.axis_index("c")
    s = lax.axis_index("s")
    lane = lax.iota(jnp.int32, 16)

    def _z(i, _):
        for j in range(d // 16):
            rows[i, pl.ds(j * 16, 16)] = jnp.zeros((16,), jnp.float32)
        return 0
    lax.fori_loop(0, _C, _z, 0)
    nfull = _RPT // _C
    for r in range(nfull):
        pltpu.sync_copy(rows, acc.at[pl.ds(s * _RPT + r * _C, _C)])
    rem = _RPT - nfull * _C
    if rem:
        pltpu.sync_copy(rows.at[pl.ds(0, rem)],
                        acc.at[pl.ds(s * _RPT + nfull * _C, rem)])
    plsc.subcore_barrier()

    base = (c * 16 + s) * _EPT

    def _chunk(t, _):
        off = base + t * _C
        pltpu.sync_copy(rowh.at[pl.ds(off, _C)], rowv)
        pltpu.sync_copy(colh.at[pl.ds(off, _C)], colv)
        for j in range(_C // 16):
            pos = t * _C + j * 16 + lane
            valid = pos < _EPT
            rv = rowv[pl.ds(j * 16, 16)]
            cv = colv[pl.ds(j * 16, 16)]
            rowv[pl.ds(j * 16, 16)] = jnp.where(valid, rv, _N)
            colv[pl.ds(j * 16, 16)] = jnp.where(valid, cv, 0)
        pltpu.async_copy(vals.at[colv], rows, sem).wait()
        pltpu.sync_copy(rows, acc.at[rowv], add=True)
        return 0

    lax.fori_loop(0, _NCHUNK, _chunk, 0)
    plsc.subcore_barrier()
    pltpu.sync_copy(acc.at[pl.ds(s * _RPT, _RPT)],
                    out.at[c, pl.ds(s * _RPT, _RPT)])


@functools.lru_cache(maxsize=None)
def _edgepass_fn(d):
    return pl.kernel(
        _edgepass_body,
        out_type=jax.ShapeDtypeStruct((2, _NPAD, d), jnp.float32),
        mesh=_mesh(),
        scratch_types=[
            pltpu.VMEM((_C,), jnp.int32),
            pltpu.VMEM((_C,), jnp.int32),
            pltpu.VMEM((_C, d), jnp.float32),
            pltpu.VMEM_SHARED((_NPAD, d), jnp.float32),
            pltpu.SemaphoreType.DMA,
        ],
    )


def _edgepass(vals, rowp, colp):
    return _edgepass_fn(vals.shape[1])(vals, rowp, colp)


def _lanesum(v16):
    return plsc.all_reduce_population_count(v16 > 0) * 0 + v16


_NG = _NPAD // 16


def _topk1_body(sh, mph, bh, meta, ufull, mpfull, bvfull, ubuf,
                mrow, gidx, sem):
    del sem
    c = lax.axis_index("c")
    s = lax.axis_index("s")
    w = c * 16 + s
    lane = lax.iota(jnp.int32, 16)
    zi = jnp.zeros((16,), jnp.int32)

    pltpu.sync_copy(sh, ufull)
    pltpu.sync_copy(mph, mpfull)
    pltpu.sync_copy(bh, bvfull)

    for gi in range(2):
        g = 2 * w + gi
        g16 = zi + g

        def _mask(i, acc):
            u = plsc.bitcast(ufull[i, pl.ds(0, 16)], jnp.int32)
            valid = ((bvfull[i, pl.ds(0, 16)] == g16)
                     & (mpfull[i, pl.ds(0, 16)] > 0))
            ubuf[i, pl.ds(0, 16)] = jnp.where(valid, u, -1)
            return acc + jnp.where(valid, 1, 0)
        cnt16 = _lanesum(lax.fori_loop(0, _NG, _mask, zi))
        k16 = lax.shift_right_logical(cnt16 + 1, 1)

        def _count_ge(cand16):
            def _grp(i, acc):
                v = ubuf[i, pl.ds(0, 16)]
                return acc + jnp.where(v >= cand16, 1, 0)
            return _lanesum(lax.fori_loop(0, _NG, _grp, zi))

        def _bit(bi, thr16):
            sh16 = zi + (30 - bi)
            cand16 = thr16 + lax.shift_left(zi + 1, sh16)
            cge16 = _count_ge(cand16)
            return jnp.where(cge16 >= k16, cand16, thr16)
        thr16 = lax.fori_loop(0, 31, _bit, zi)
        cgt16 = _count_ge(thr16 + 1)
        r16 = k16 - cgt16

        def _cnt_tie_lt(bb16):
            def _grp(i, acc):
                v = ubuf[i, pl.ds(0, 16)]
                pos = i * 16 + lane
                return acc + jnp.where((v == thr16) & (pos < bb16), 1, 0)
            return _lanesum(lax.fori_loop(0, _NG, _grp, zi))

        def _bbit(bi, b16):
            cand16 = b16 + lax.shift_left(zi + 1, zi + (13 - bi))
            c16 = _cnt_tie_lt(cand16)
            return jnp.where(c16 <= r16, cand16, b16)
        b16 = lax.fori_loop(0, 14, _bbit, zi)

        mrow[0, pl.ds(0, 16)] = out16
        for j in range(1, 8):
            mrow[0, pl.ds(j * 16, 16)] = jnp.zeros((16,), jnp.int32)
        gidx[pl.ds(0, 16)] = jnp.where(lane == 0, zi + g, zi + (_G + w))
        pltpu.sync_copy(mrow, meta.at[gidx])


@functools.lru_cache(maxsize=None)
def _topk1_fn():
    return pl.kernel(
        _topk1_body,
        out_type=jax.ShapeDtypeStruct((_G + 32, _D), jnp.int32),
        mesh=_mesh(),
        scratch_types=[
            pltpu.VMEM((_NG, 16), jnp.float32),
            pltpu.VMEM((_NG, 16), jnp.float32),
            pltpu.VMEM((_NG, 16), jnp.int32),
            pltpu.VMEM((_NG, 16), jnp.int32),
            pltpu.VMEM((16, _D), jnp.int32),
            pltpu.VMEM((16,), jnp.int32),
            pltpu.SemaphoreType.DMA,
        ],
    )


def _topk1(s2d, mp2d, b2d):
    return _topk1_fn()(s2d, mp2d, b2d)


def _topk2_body(sh, mph, bh, metah, hh, mout, xmax, xsum,
                ufull, mpfull, bvfull, metl, mbuf,
                hbuf, orow, gidx, sem):
    del sem
    c = lax.axis_index("c")
    s = lax.axis_index("s")
    w = c * 16 + s
    lane = lax.iota(jnp.int32, 16)
    z16i = jnp.zeros((16,), jnp.int32)
    one16i = z16i + 1

    pltpu.sync_copy(metah, metl)
    pltpu.sync_copy(sh, ufull)
    pltpu.sync_copy(mph, mpfull)
    pltpu.sync_copy(bh, bvfull)

    for i in range(_NPT // 16):
        r = w * (_NPT // 16) + i
        g16 = bvfull[r, pl.ds(0, 16)]
        thr = plsc.load_gather(metl, [g16, z16i])
        bbv = plsc.load_gather(metl, [g16, one16i])
        u = plsc.bitcast(ufull[r, pl.ds(0, 16)], jnp.int32)
        mp16 = mpfull[r, pl.ds(0, 16)]
        pos = r * 16 + lane
        sel = (mp16 > 0) & ((u > thr) | ((u == thr) & (pos < bbv)))
        mbuf[0, pl.ds(i * 16, 16)] = jnp.where(sel, 1.0, 0.0)
    gidx[pl.ds(0, 16)] = jnp.where(lane == 0, z16i + w, z16i + (32 + w))
    pltpu.sync_copy(mbuf, mout.at[gidx])

    zv = jnp.zeros((16,), jnp.float32)
    for gi in range(2):
        g = 2 * w + gi
        g16 = z16i + g
        thr16 = plsc.load_gather(metl, [g16, z16i])
        b16 = plsc.load_gather(metl, [g16, one16i])

        def _chunk(t, carry):
            off = pl.multiple_of(t * _C, 8)
            pltpu.sync_copy(hh.at[pl.ds(off, _C)], hbuf)

            def _row(rr, cr):
                o = t * _C + rr
                orow16 = z16i + lax.shift_right_logical(o, 4)
                ocol16 = z16i + lax.bitwise_and(o, 15)
                pos16 = z16i + o
                u16 = plsc.bitcast(
                    plsc.load_gather(ufull, [orow16, ocol16]), jnp.int32)
                mp16 = plsc.load_gather(mpfull, [orow16, ocol16])
                bg16 = plsc.load_gather(bvfull, [orow16, ocol16])
                selb = ((bg16 == g16) & (mp16 > 0)
                        & ((u16 > thr16) | ((u16 == thr16) & (pos16 < b16))))
                wr = jnp.where(selb, jnp.float32(1.0), jnp.float32(0.0))
                nmax = []
                nsum = []
                for j in range(8):
                    hv = hbuf[rr, pl.ds(j * 16, 16)] * wr
                    nmax.append(jnp.maximum(cr[j], hv))
                    nsum.append(cr[8 + j] + hv)
                return tuple(nmax) + tuple(nsum)
            return lax.fori_loop(0, _C, _row, carry)

        init = tuple(zv for _ in range(16))
        res = lax.fori_loop(0, _NPAD // _C, _chunk, init)
        gidx[pl.ds(0, 16)] = jnp.where(lane == 0, z16i + g, z16i + (_G + w))
        for j in range(8):
            orow[0, pl.ds(j * 16, 16)] = res[j]
        pltpu.sync_copy(orow, xmax.at[gidx])
        for j in range(8):
            orow[0, pl.ds(j * 16, 16)] = res[8 + j]
        pltpu.sync_copy(orow, xsum.at[gidx])


@functools.lru_cache(maxsize=None)
def _topk2_fn():
    return pl.kernel(
        _topk2_body,
        out_type=(
            jax.ShapeDtypeStruct((64, _NPT), jnp.float32),
            jax.ShapeDtypeStruct((_G + 32, _D), jnp.float32),
            jax.ShapeDtypeStruct((_G + 32, _D), jnp.float32),
        ),
        mesh=_mesh(),
        scratch_types=[
            pltpu.VMEM((_NG, 16), jnp.float32),
            pltpu.VMEM((_NG, 16), jnp.float32),
            pltpu.VMEM((_NG, 16), jnp.int32),
            pltpu.VMEM((_G + 32, _D), jnp.int32),
            pltpu.VMEM((16, _NPT), jnp.float32),
            pltpu.VMEM((_C, _D), jnp.float32),
            pltpu.VMEM((16, _D), jnp.float32),
            pltpu.VMEM((16,), jnp.int32),
            pltpu.SemaphoreType.DMA,
        ],
    )


def _topk2(s2d, mp2d, b2d, meta, hpad):
    return _topk2_fn()(s2d, mp2d, b2d, meta, hpad)


_BLK = 1000


def _scales(cvec, mvec):
    degs = mvec * cvec + mvec
    a = mvec * jnp.where(degs > 0, lax.rsqrt(jnp.maximum(degs, 1e-30)), 0.0)
    degi = mvec * cvec
    av = mvec * jnp.where(degi > 0, lax.rsqrt(jnp.maximum(degi, 1e-30)), 0.0)
    return a, av


def _mm_body(x_ref, w_ref, c_ref, m_ref, xw_ref, v_ref):
    xw = jnp.dot(x_ref[...], w_ref[...], preferred_element_type=jnp.float32)
    a, _ = _scales(c_ref[...], m_ref[...])
    xw_ref[...] = xw
    v_ref[...] = xw * a


def _mm(x, W, cvec, mvec):
    n = x.shape[0]
    grid = n // _BLK
    return pl.pallas_call(
        _mm_body,
        grid=(grid,),
        in_specs=[
            pl.BlockSpec((_BLK, _D), lambda i: (i, 0)),
            pl.BlockSpec((_D, _D), lambda i: (0, 0)),
            pl.BlockSpec((_BLK, 1), lambda i: (i, 0)),
            pl.BlockSpec((_BLK, 1), lambda i: (i, 0)),
        ],
        out_specs=(
            pl.BlockSpec((_BLK, _D), lambda i: (i, 0)),
            pl.BlockSpec((_BLK, _D), lambda i: (i, 0)),
        ),
        out_shape=(
            jax.ShapeDtypeStruct((n, _D), jnp.float32),
            jax.ShapeDtypeStruct((n, _D), jnp.float32),
        ),
    )(x, W, cvec, mvec)


def _h_body(p0_ref, p1_ref, xw_ref, c_ref, m_ref, b_ref, h_ref, hv_ref):
    a, av = _scales(c_ref[...], m_ref[...])
    h = jax.nn.relu((p0_ref[...] + p1_ref[...]) * a
                    + xw_ref[...] * a * a + b_ref[...])
    h_ref[...] = h
    hv_ref[...] = h * av


def _h_stage(p0, p1, xw, cvec, mvec, b):
    n = xw.shape[0]
    grid = n // _BLK
    return pl.pallas_call(
        _h_body,
        grid=(grid,),
        in_specs=[
            pl.BlockSpec((_BLK, _D), lambda i: (i, 0)),
            pl.BlockSpec((_BLK, _D), lambda i: (i, 0)),
            pl.BlockSpec((_BLK, _D), lambda i: (i, 0)),
            pl.BlockSpec((_BLK, 1), lambda i: (i, 0)),
            pl.BlockSpec((_BLK, 1), lambda i: (i, 0)),
            pl.BlockSpec((1, _D), lambda i: (0, 0)),
        ],
        out_specs=(
            pl.BlockSpec((_BLK, _D), lambda i: (i, 0)),
            pl.BlockSpec((_BLK, _D), lambda i: (i, 0)),
        ),
        out_shape=(
            jax.ShapeDtypeStruct((n, _D), jnp.float32),
            jax.ShapeDtypeStruct((n, _D), jnp.float32),
        ),
    )(p0, p1, xw, cvec, mvec, b)


def _s_body(h_ref, q0_ref, q1_ref, c_ref, m_ref, s_ref):
    _, av = _scales(c_ref[...], m_ref[...])
    s_ref[...] = jnp.sum(
        jnp.abs(h_ref[...] - (q0_ref[...] + q1_ref[...]) * av),
        axis=1, keepdims=True)


def _s_stage(h, q0, q1, cvec, mvec):
    n = h.shape[0]
    grid = n // _BLK
    return pl.pallas_call(
        _s_body,
        grid=(grid,),
        in_specs=[
            pl.BlockSpec((_BLK, _D), lambda i: (i, 0)),
            pl.BlockSpec((_BLK, _D), lambda i: (i, 0)),
            pl.BlockSpec((_BLK, _D), lambda i: (i, 0)),
            pl.BlockSpec((_BLK, 1), lambda i: (i, 0)),
            pl.BlockSpec((_BLK, 1), lambda i: (i, 0)),
        ],
        out_specs=pl.BlockSpec((_BLK, 1), lambda i: (i, 0)),
        out_shape=jax.ShapeDtypeStruct((n, 1), jnp.float32),
    )(h, q0, q1, cvec, mvec)


def _out_body(x1m, x1s, c1, x2m, x2s, x3m, x3s, c2, w_ref, b_ref, o_ref):
    def xcat(xm, xs, cc):
        return jnp.concatenate(
            [jax.nn.relu(xm[...]),
             jax.nn.relu(xs[...] / jnp.maximum(cc[...], 1.0))], axis=1)
    xc = (xcat(x1m, x1s, c1) + xcat(x2m, x2s, c2) + xcat(x3m, x3s, c2))
    o_ref[...] = jnp.dot(xc, w_ref[...],
                         preferred_element_type=jnp.float32) + b_ref[...]


def _out_stage(x1m, x1s, c1, x2m, x2s, x3m, x3s, c2, linW, linb):
    return pl.pallas_call(
        _out_body,
        out_shape=jax.ShapeDtypeStruct((_G, linW.shape[1]), jnp.float32),
    )(x1m, x1s, c1, x2m, x2s, x3m, x3s, c2, linW, linb.reshape(1, -1))


def _topk_m(score, batch, nmask, g):
    n = score.shape[0]
    cnt_all = jnp.zeros((g,), jnp.int32).at[batch].add(1)
    cnt_valid = jnp.zeros((g,), jnp.int32).at[batch].add(
        nmask.astype(jnp.int32))
    k = jnp.ceil(0.5 * cnt_valid.astype(jnp.float32)).astype(jnp.int32)
    key = jnp.where(nmask, -score, jnp.inf)
    order = jnp.lexsort((key, batch))
    starts = jnp.cumsum(cnt_all) - cnt_all
    bo = batch[order]
    rank = jnp.arange(n, dtype=jnp.int32) - starts[bo]
    sel = (rank < k[bo]) & nmask[order]
    m = jnp.zeros((n,), bool).at[order].set(sel)
    return m, k


def _pool(h, m, batch, g):
    hm = jnp.where(m[:, None], h, 0.0)
    xmax = jax.ops.segment_max(hm, batch, num_segments=g)
    xsum = jax.ops.segment_sum(hm, batch, num_segments=g)
    return xmax, xsum


def kernel(x, edge_index, batch, W1, b1, W2, b2, W3, b3, linW, linb):
    n = x.shape[0]
    g = _G
    f32 = x.dtype
    row = edge_index[1]
    col = edge_index[0]
    epad = _NCHUNK * _C * 32 - _E
    rowp = jnp.concatenate([row, jnp.zeros((epad,), row.dtype)])
    colp = jnp.concatenate([col, jnp.zeros((epad,), col.dtype)])
    ones_col = jnp.ones((n, 1), f32)

    def agg(v):
        p = _edgepass(v, rowp, colp)
        return p[0, :n], p[1, :n]

    i0, i1 = agg(jnp.ones((n, _D), f32))
    indeg = (i0[:, :1] + i1[:, :1])

    xw1, v1 = _mm(x, W1, indeg, ones_col)
    p0, p1 = agg(v1)
    h1, hv1 = _h_stage(p0, p1, xw1, indeg, ones_col, b1.reshape(1, -1))
    q0, q1 = agg(hv1)
    s1 = _s_stage(h1, q0, q1, indeg, ones_col)[:, 0]

    m1, k1 = _topk_m(s1, batch, jnp.ones((n,), bool), g)
    m1p = m1.astype(f32)
    cnt1 = k1[:, None].astype(f32)
    x1m, x1s = _pool(h1, m1, batch, g)

    c0, c1 = agg(jnp.broadcast_to(m1p[:, None], (n, _D)))
    c2v = c0[:, :1] + c1[:, :1]
    m1c = m1p[:, None]

    xw2, v2 = _mm(h1, W2, c2v, m1c)
    p0, p1 = agg(v2)
    h2, hv2 = _h_stage(p0, p1, xw2, c2v, m1c, b2.reshape(1, -1))
    q0, q1 = agg(hv2)
    s2 = _s_stage(h2, q0, q1, c2v, m1c)[:, 0]

    m2, k2 = _topk_m(s2, batch, m1, g)
    m2p = m2.astype(f32)
    cnt2 = k2[:, None].astype(f32)
    x2m, x2s = _pool(h2, m2, batch, g)

    d0, d1 = agg(jnp.broadcast_to(m2p[:, None], (n, _D)))
    c3v = d0[:, :1] + d1[:, :1]
    m2c = m2p[:, None]

    xw3, v3 = _mm(h2, W3, c3v, m2c)
    p0, p1 = agg(v3)
    h3, _ = _h_stage(p0, p1, xw3, c3v, m2c, b3.reshape(1, -1))
    x3m, x3s = _pool(h3, m2, batch, g)

    return _out_stage(x1m, x1s, cnt1, x2m, x2s, x3m, x3s, cnt2,
                      linW, linb)


# speedup vs baseline: 1.1014x; 1.1014x over previous
import functools

import jax
import jax.numpy as jnp
from jax import lax
from jax.experimental import pallas as pl
from jax.experimental.pallas import tpu as pltpu
from jax.experimental.pallas import tpu_sc as plsc

_G = 64
_N = 10000
_E = 320000
_D = 128
_NPAD = 10240
_C = 128
_EPT = _E // 32
_NCHUNK = (_EPT + _C - 1) // _C
_RPT = _NPAD // 16
_HROWS = 128
_NPT = _NPAD // 32


def _mesh():
    return plsc.VectorSubcoreMesh(core_axis_name="c", subcore_axis_name="s")


def _edgepass_body(vals, rowh, colh, out,
                   colv0, rowv0, rows0, colv1, rowv1, rows1, acc,
                   semg0, semg1, sems0, sems1):
    d = rows0.shape[1]
    c = lax.axis_index("c")
    s = lax.axis_index("s")
    lane = lax.iota(jnp.int32, 16)
    colv = [colv0, colv1]
    rowv = [rowv0, rowv1]
    rows = [rows0, rows1]
    semg = [semg0, semg1]
    sems = [sems0, sems1]

    def _z(i, _):
        for j in range(d // 16):
            rows0[i, pl.ds(j * 16, 16)] = jnp.zeros((16,), jnp.float32)
            rows1[i, pl.ds(j * 16, 16)] = jnp.zeros((16,), jnp.float32)
        return 0
    lax.fori_loop(0, _C, _z, 0)
    nfull = _RPT // _C
    for r in range(nfull):
        pltpu.sync_copy(rows0, acc.at[pl.ds(s * _RPT + r * _C, _C)])
    rem = _RPT - nfull * _C
    if rem:
        pltpu.sync_copy(rows0.at[pl.ds(0, rem)],
                        acc.at[pl.ds(s * _RPT + nfull * _C, rem)])
    plsc.subcore_barrier()

    base = (c * 16 + s) * _EPT

    def load_idx(t, bi):
        off = base + t * _C
        pltpu.sync_copy(rowh.at[pl.ds(off, _C)], rowv[bi])
        pltpu.sync_copy(colh.at[pl.ds(off, _C)], colv[bi])
        for j in range(_C // 16):
            pos = t * _C + j * 16 + lane
            valid = pos < _EPT
            rv = rowv[bi][pl.ds(j * 16, 16)]
            cv = colv[bi][pl.ds(j * 16, 16)]
            rowv[bi][pl.ds(j * 16, 16)] = jnp.where(valid, rv, _N)
            colv[bi][pl.ds(j * 16, 16)] = jnp.where(valid, cv, 0)

    def start_gather(bi):
        pltpu.async_copy(vals.at[colv[bi]], rows[bi], semg[bi])

    def wait_gather(bi):
        pltpu.make_async_copy(vals.at[colv[bi]], rows[bi], semg[bi]).wait()

    def start_scatter(bi):
        pltpu.async_copy(rows[bi], acc.at[rowv[bi]], sems[bi], add=True)

    def wait_scatter(bi):
        pltpu.make_async_copy(rows[bi], acc.at[rowv[bi]], sems[bi]).wait()

    load_idx(0, 0)
    start_gather(0)
    for j in range(_C // 16):
        rowv[1][pl.ds(j * 16, 16)] = jnp.zeros((16,), jnp.int32) + _N
    start_scatter(1)

    def _pair(p, _):
        t1 = 2 * p + 1
        t2 = 2 * p + 2
        wait_gather(0)
        wait_scatter(1)
        load_idx(t1, 1)
        start_gather(1)
        start_scatter(0)
        wait_gather(1)
        wait_scatter(0)
        load_idx(t2, 0)
        start_gather(0)
        start_scatter(1)
        return 0
    assert _NCHUNK == 79
    lax.fori_loop(0, (_NCHUNK - 1) // 2, _pair, 0)
    wait_gather(0)
    wait_scatter(1)
    start_scatter(0)
    wait_scatter(0)

    plsc.subcore_barrier()
    pltpu.sync_copy(acc.at[pl.ds(s * _RPT, _RPT)],
                    out.at[c, pl.ds(s * _RPT, _RPT)])


@functools.lru_cache(maxsize=None)
def _edgepass_fn(d):
    return pl.kernel(
        _edgepass_body,
        out_type=jax.ShapeDtypeStruct((2, _NPAD, d), jnp.float32),
        mesh=_mesh(),
        scratch_types=[
            pltpu.VMEM((_C,), jnp.int32),
            pltpu.VMEM((_C,), jnp.int32),
            pltpu.VMEM((_C, d), jnp.float32),
            pltpu.VMEM((_C,), jnp.int32),
            pltpu.VMEM((_C,), jnp.int32),
            pltpu.VMEM((_C, d), jnp.float32),
            pltpu.VMEM_SHARED((_NPAD, d), jnp.float32),
            pltpu.SemaphoreType.DMA,
            pltpu.SemaphoreType.DMA,
            pltpu.SemaphoreType.DMA,
            pltpu.SemaphoreType.DMA,
        ],
    )


def _edgepass(vals, rowp, colp):
    return _edgepass_fn(vals.shape[1])(vals, rowp, colp)


def _lanesum(v16):
    return plsc.all_reduce_population_count(v16 > 0) * 0 + v16


_NG = _NPAD // 16


def _topk1_body(sh, mph, bh, meta, ufull, mpfull, bvfull, ubuf,
                mrow, gidx, sem):
    del sem
    c = lax.axis_index("c")
    s = lax.axis_index("s")
    w = c * 16 + s
    lane = lax.iota(jnp.int32, 16)
    zi = jnp.zeros((16,), jnp.int32)

    pltpu.sync_copy(sh, ufull)
    pltpu.sync_copy(mph, mpfull)
    pltpu.sync_copy(bh, bvfull)

    for gi in range(2):
        g = 2 * w + gi
        g16 = zi + g

        def _mask(i, acc):
            u = plsc.bitcast(ufull[i, pl.ds(0, 16)], jnp.int32)
            valid = ((bvfull[i, pl.ds(0, 16)] == g16)
                     & (mpfull[i, pl.ds(0, 16)] > 0))
            ubuf[i, pl.ds(0, 16)] = jnp.where(valid, u, -1)
            return acc + jnp.where(valid, 1, 0)
        cnt16 = _lanesum(lax.fori_loop(0, _NG, _mask, zi))
        k16 = lax.shift_right_logical(cnt16 + 1, 1)

        def _count_ge(cand16):
            def _grp(i, acc):
                v = ubuf[i, pl.ds(0, 16)]
                return acc + jnp.where(v >= cand16, 1, 0)
            return _lanesum(lax.fori_loop(0, _NG, _grp, zi))

        def _bit(bi, thr16):
            sh16 = zi + (30 - bi)
            cand16 = thr16 + lax.shift_left(zi + 1, sh16)
            cge16 = _count_ge(cand16)
            return jnp.where(cge16 >= k16, cand16, thr16)
        thr16 = lax.fori_loop(0, 31, _bit, zi)
        cgt16 = _count_ge(thr16 + 1)
        r16 = k16 - cgt16

        def _cnt_tie_lt(bb16):
            def _grp(i, acc):
                v = ubuf[i, pl.ds(0, 16)]
                pos = i * 16 + lane
                return acc + jnp.where((v == thr16) & (pos < bb16), 1, 0)
            return _lanesum(lax.fori_loop(0, _NG, _grp, zi))

        def _bbit(bi, b16):
            cand16 = b16 + lax.shift_left(zi + 1, zi + (13 - bi))
            c16 = _cnt_tie_lt(cand16)
            return jnp.where(c16 <= r16, cand16, b16)
        b16 = lax.fori_loop(0, 14, _bbit, zi)

        mrow[0, pl.ds(0, 16)] = out16
        for j in range(1, 8):
            mrow[0, pl.ds(j * 16, 16)] = jnp.zeros((16,), jnp.int32)
        gidx[pl.ds(0, 16)] = jnp.where(lane == 0, zi + g, zi + (_G + w))
        pltpu.sync_copy(mrow, meta.at[gidx])


@functools.lru_cache(maxsize=None)
def _topk1_fn():
    return pl.kernel(
        _topk1_body,
        out_type=jax.ShapeDtypeStruct((_G + 32, _D), jnp.int32),
        mesh=_mesh(),
        scratch_types=[
            pltpu.VMEM((_NG, 16), jnp.float32),
            pltpu.VMEM((_NG, 16), jnp.float32),
            pltpu.VMEM((_NG, 16), jnp.int32),
            pltpu.VMEM((_NG, 16), jnp.int32),
            pltpu.VMEM((16, _D), jnp.int32),
            pltpu.VMEM((16,), jnp.int32),
            pltpu.SemaphoreType.DMA,
        ],
    )


def _topk1(s2d, mp2d, b2d):
    return _topk1_fn()(s2d, mp2d, b2d)


def _topk2_body(sh, mph, bh, metah, hh, mout, xmax, xsum,
                ufull, mpfull, bvfull, metl, mbuf,
                hbuf, orow, gidx, sem):
    del sem
    c = lax.axis_index("c")
    s = lax.axis_index("s")
    w = c * 16 + s
    lane = lax.iota(jnp.int32, 16)
    z16i = jnp.zeros((16,), jnp.int32)
    one16i = z16i + 1

    pltpu.sync_copy(metah, metl)
    pltpu.sync_copy(sh, ufull)
    pltpu.sync_copy(mph, mpfull)
    pltpu.sync_copy(bh, bvfull)

    for i in range(_NPT // 16):
        r = w * (_NPT // 16) + i
        g16 = bvfull[r, pl.ds(0, 16)]
        thr = plsc.load_gather(metl, [g16, z16i])
        bbv = plsc.load_gather(metl, [g16, one16i])
        u = plsc.bitcast(ufull[r, pl.ds(0, 16)], jnp.int32)
        mp16 = mpfull[r, pl.ds(0, 16)]
        pos = r * 16 + lane
        sel = (mp16 > 0) & ((u > thr) | ((u == thr) & (pos < bbv)))
        mbuf[0, pl.ds(i * 16, 16)] = jnp.where(sel, 1.0, 0.0)
    gidx[pl.ds(0, 16)] = jnp.where(lane == 0, z16i + w, z16i + (32 + w))
    pltpu.sync_copy(mbuf, mout.at[gidx])

    zv = jnp.zeros((16,), jnp.float32)
    for gi in range(2):
        g = 2 * w + gi
        g16 = z16i + g
        thr16 = plsc.load_gather(metl, [g16, z16i])
        b16 = plsc.load_gather(metl, [g16, one16i])

        def _chunk(t, carry):
            off = pl.multiple_of(t * _C, 8)
            pltpu.sync_copy(hh.at[pl.ds(off, _C)], hbuf)

            def _row(rr, cr):
                o = t * _C + rr
                orow16 = z16i + lax.shift_right_logical(o, 4)
                ocol16 = z16i + lax.bitwise_and(o, 15)
                pos16 = z16i + o
                u16 = plsc.bitcast(
                    plsc.load_gather(ufull, [orow16, ocol16]), jnp.int32)
                mp16 = plsc.load_gather(mpfull, [orow16, ocol16])
                bg16 = plsc.load_gather(bvfull, [orow16, ocol16])
                selb = ((bg16 == g16) & (mp16 > 0)
                        & ((u16 > thr16) | ((u16 == thr16) & (pos16 < b16))))
                wr = jnp.where(selb, jnp.float32(1.0), jnp.float32(0.0))
                nmax = []
                nsum = []
                for j in range(8):
                    hv = hbuf[rr, pl.ds(j * 16, 16)] * wr
                    nmax.append(jnp.maximum(cr[j], hv))
                    nsum.append(cr[8 + j] + hv)
                return tuple(nmax) + tuple(nsum)
            return lax.fori_loop(0, _C, _row, carry)

        init = tuple(zv for _ in range(16))
        res = lax.fori_loop(0, _NPAD // _C, _chunk, init)
        gidx[pl.ds(0, 16)] = jnp.where(lane == 0, z16i + g, z16i + (_G + w))
        for j in range(8):
            orow[0, pl.ds(j * 16, 16)] = res[j]
        pltpu.sync_copy(orow, xmax.at[gidx])
        for j in range(8):
            orow[0, pl.ds(j * 16, 16)] = res[8 + j]
        pltpu.sync_copy(orow, xsum.at[gidx])


@functools.lru_cache(maxsize=None)
def _topk2_fn():
    return pl.kernel(
        _topk2_body,
        out_type=(
            jax.ShapeDtypeStruct((64, _NPT), jnp.float32),
            jax.ShapeDtypeStruct((_G + 32, _D), jnp.float32),
            jax.ShapeDtypeStruct((_G + 32, _D), jnp.float32),
        ),
        mesh=_mesh(),
        scratch_types=[
            pltpu.VMEM((_NG, 16), jnp.float32),
            pltpu.VMEM((_NG, 16), jnp.float32),
            pltpu.VMEM((_NG, 16), jnp.int32),
            pltpu.VMEM((_G + 32, _D), jnp.int32),
            pltpu.VMEM((16, _NPT), jnp.float32),
            pltpu.VMEM((_C, _D), jnp.float32),
            pltpu.VMEM((16, _D), jnp.float32),
            pltpu.VMEM((16,), jnp.int32),
            pltpu.SemaphoreType.DMA,
        ],
    )


def _topk2(s2d, mp2d, b2d, meta, hpad):
    return _topk2_fn()(s2d, mp2d, b2d, meta, hpad)


_BLK = 1000


def _scales(cvec, mvec):
    degs = mvec * cvec + mvec
    a = mvec * jnp.where(degs > 0, lax.rsqrt(jnp.maximum(degs, 1e-30)), 0.0)
    degi = mvec * cvec
    av = mvec * jnp.where(degi > 0, lax.rsqrt(jnp.maximum(degi, 1e-30)), 0.0)
    return a, av


def _mm_body(x_ref, w_ref, c_ref, m_ref, xw_ref, v_ref):
    xw = jnp.dot(x_ref[...], w_ref[...], preferred_element_type=jnp.float32)
    a, _ = _scales(c_ref[...], m_ref[...])
    xw_ref[...] = xw
    v_ref[...] = xw * a


def _mm(x, W, cvec, mvec):
    n = x.shape[0]
    grid = n // _BLK
    return pl.pallas_call(
        _mm_body,
        grid=(grid,),
        in_specs=[
            pl.BlockSpec((_BLK, _D), lambda i: (i, 0)),
            pl.BlockSpec((_D, _D), lambda i: (0, 0)),
            pl.BlockSpec((_BLK, 1), lambda i: (i, 0)),
            pl.BlockSpec((_BLK, 1), lambda i: (i, 0)),
        ],
        out_specs=(
            pl.BlockSpec((_BLK, _D), lambda i: (i, 0)),
            pl.BlockSpec((_BLK, _D), lambda i: (i, 0)),
        ),
        out_shape=(
            jax.ShapeDtypeStruct((n, _D), jnp.float32),
            jax.ShapeDtypeStruct((n, _D), jnp.float32),
        ),
    )(x, W, cvec, mvec)


def _h_body(p0_ref, p1_ref, xw_ref, c_ref, m_ref, b_ref, h_ref, hv_ref):
    a, av = _scales(c_ref[...], m_ref[...])
    h = jax.nn.relu((p0_ref[...] + p1_ref[...]) * a
                    + xw_ref[...] * a * a + b_ref[...])
    h_ref[...] = h
    hv_ref[...] = h * av


def _h_stage(p0, p1, xw, cvec, mvec, b):
    n = xw.shape[0]
    grid = n // _BLK
    return pl.pallas_call(
        _h_body,
        grid=(grid,),
        in_specs=[
            pl.BlockSpec((_BLK, _D), lambda i: (i, 0)),
            pl.BlockSpec((_BLK, _D), lambda i: (i, 0)),
            pl.BlockSpec((_BLK, _D), lambda i: (i, 0)),
            pl.BlockSpec((_BLK, 1), lambda i: (i, 0)),
            pl.BlockSpec((_BLK, 1), lambda i: (i, 0)),
            pl.BlockSpec((1, _D), lambda i: (0, 0)),
        ],
        out_specs=(
            pl.BlockSpec((_BLK, _D), lambda i: (i, 0)),
            pl.BlockSpec((_BLK, _D), lambda i: (i, 0)),
        ),
        out_shape=(
            jax.ShapeDtypeStruct((n, _D), jnp.float32),
            jax.ShapeDtypeStruct((n, _D), jnp.float32),
        ),
    )(p0, p1, xw, cvec, mvec, b)


def _s_body(h_ref, q0_ref, q1_ref, c_ref, m_ref, s_ref):
    _, av = _scales(c_ref[...], m_ref[...])
    s_ref[...] = jnp.sum(
        jnp.abs(h_ref[...] - (q0_ref[...] + q1_ref[...]) * av),
        axis=1, keepdims=True)


def _s_stage(h, q0, q1, cvec, mvec):
    n = h.shape[0]
    grid = n // _BLK
    return pl.pallas_call(
        _s_body,
        grid=(grid,),
        in_specs=[
            pl.BlockSpec((_BLK, _D), lambda i: (i, 0)),
            pl.BlockSpec((_BLK, _D), lambda i: (i, 0)),
            pl.BlockSpec((_BLK, _D), lambda i: (i, 0)),
            pl.BlockSpec((_BLK, 1), lambda i: (i, 0)),
            pl.BlockSpec((_BLK, 1), lambda i: (i, 0)),
        ],
        out_specs=pl.BlockSpec((_BLK, 1), lambda i: (i, 0)),
        out_shape=jax.ShapeDtypeStruct((n, 1), jnp.float32),
    )(h, q0, q1, cvec, mvec)


def _out_body(x1m, x1s, c1, x2m, x2s, x3m, x3s, c2, w_ref, b_ref, o_ref):
    def xcat(xm, xs, cc):
        return jnp.concatenate(
            [jax.nn.relu(xm[...]),
             jax.nn.relu(xs[...] / jnp.maximum(cc[...], 1.0))], axis=1)
    xc = (xcat(x1m, x1s, c1) + xcat(x2m, x2s, c2) + xcat(x3m, x3s, c2))
    o_ref[...] = jnp.dot(xc, w_ref[...],
                         preferred_element_type=jnp.float32) + b_ref[...]


def _out_stage(x1m, x1s, c1, x2m, x2s, x3m, x3s, c2, linW, linb):
    return pl.pallas_call(
        _out_body,
        out_shape=jax.ShapeDtypeStruct((_G, linW.shape[1]), jnp.float32),
    )(x1m, x1s, c1, x2m, x2s, x3m, x3s, c2, linW, linb.reshape(1, -1))


def _topk_m(score, batch, nmask, g):
    n = score.shape[0]
    cnt_all = jnp.zeros((g,), jnp.int32).at[batch].add(1)
    cnt_valid = jnp.zeros((g,), jnp.int32).at[batch].add(
        nmask.astype(jnp.int32))
    k = jnp.ceil(0.5 * cnt_valid.astype(jnp.float32)).astype(jnp.int32)
    key = jnp.where(nmask, -score, jnp.inf)
    order = jnp.lexsort((key, batch))
    starts = jnp.cumsum(cnt_all) - cnt_all
    bo = batch[order]
    rank = jnp.arange(n, dtype=jnp.int32) - starts[bo]
    sel = (rank < k[bo]) & nmask[order]
    m = jnp.zeros((n,), bool).at[order].set(sel)
    return m, k


def _pool(h, m, batch, g):
    hm = jnp.where(m[:, None], h, 0.0)
    xmax = jax.ops.segment_max(hm, batch, num_segments=g)
    xsum = jax.ops.segment_sum(hm, batch, num_segments=g)
    return xmax, xsum


def kernel(x, edge_index, batch, W1, b1, W2, b2, W3, b3, linW, linb):
    n = x.shape[0]
    g = _G
    f32 = x.dtype
    row = edge_index[1]
    col = edge_index[0]
    epad = _NCHUNK * _C * 32 - _E
    rowp = jnp.concatenate([row, jnp.zeros((epad,), row.dtype)])
    colp = jnp.concatenate([col, jnp.zeros((epad,), col.dtype)])
    ones_col = jnp.ones((n, 1), f32)

    def agg(v):
        p = _edgepass(v, rowp, colp)
        return p[0, :n], p[1, :n]

    i0, i1 = agg(jnp.ones((n, _D), f32))
    indeg = (i0[:, :1] + i1[:, :1])

    xw1, v1 = _mm(x, W1, indeg, ones_col)
    p0, p1 = agg(v1)
    h1, hv1 = _h_stage(p0, p1, xw1, indeg, ones_col, b1.reshape(1, -1))
    q0, q1 = agg(hv1)
    s1 = _s_stage(h1, q0, q1, indeg, ones_col)[:, 0]

    m1, k1 = _topk_m(s1, batch, jnp.ones((n,), bool), g)
    m1p = m1.astype(f32)
    cnt1 = k1[:, None].astype(f32)
    x1m, x1s = _pool(h1, m1, batch, g)

    c0, c1 = agg(jnp.broadcast_to(m1p[:, None], (n, _D)))
    c2v = c0[:, :1] + c1[:, :1]
    m1c = m1p[:, None]

    xw2, v2 = _mm(h1, W2, c2v, m1c)
    p0, p1 = agg(v2)
    h2, hv2 = _h_stage(p0, p1, xw2, c2v, m1c, b2.reshape(1, -1))
    q0, q1 = agg(hv2)
    s2 = _s_stage(h2, q0, q1, c2v, m1c)[:, 0]

    m2, k2 = _topk_m(s2, batch, m1, g)
    m2p = m2.astype(f32)
    cnt2 = k2[:, None].astype(f32)
    x2m, x2s = _pool(h2, m2, batch, g)

    d0, d1 = agg(jnp.broadcast_to(m2p[:, None], (n, _D)))
    c3v = d0[:, :1] + d1[:, :1]
    m2c = m2p[:, None]

    xw3, v3 = _mm(h2, W3, c3v, m2c)
    p0, p1 = agg(v3)
    h3, _ = _h_stage(p0, p1, xw3, c3v, m2c, b3.reshape(1, -1))
    x3m, x3s = _pool(h3, m2, batch, g)

    return _out_stage(x1m, x1s, cnt1, x2m, x2s, x3m, x3s, cnt2,
                      linW, linb)


# speedup vs baseline: 1.1014x; 1.0000x over previous
import functools

import jax
import jax.numpy as jnp
from jax import lax
from jax.experimental import pallas as pl
from jax.experimental.pallas import tpu as pltpu
from jax.experimental.pallas import tpu_sc as plsc

_G = 64
_N = 10000
_E = 320000
_D = 128
_NPAD = 10240
_C = 128
_EPT = _E // 32
_NCHUNK = (_EPT + _C - 1) // _C
_RPT = _NPAD // 16
_HROWS = 128
_NPT = _NPAD // 32


def _mesh():
    return plsc.VectorSubcoreMesh(core_axis_name="c", subcore_axis_name="s")


def _edgepass_body(vals, rowh, colh, out,
                   colv0, rowv0, rows0, colv1, rowv1, rows1, acc,
                   semg0, semg1, sems0, sems1):
    d = rows0.shape[1]
    c = lax.axis_index("c")
    s = lax.axis_index("s")
    lane = lax.iota(jnp.int32, 16)
    colv = [colv0, colv1]
    rowv = [rowv0, rowv1]
    rows = [rows0, rows1]
    semg = [semg0, semg1]
    sems = [sems0, sems1]

    def _z(i, _):
        for j in range(d // 16):
            rows0[i, pl.ds(j * 16, 16)] = jnp.zeros((16,), jnp.float32)
            rows1[i, pl.ds(j * 16, 16)] = jnp.zeros((16,), jnp.float32)
        return 0
    lax.fori_loop(0, _C, _z, 0)
    nfull = _RPT // _C
    for r in range(nfull):
        pltpu.sync_copy(rows0, acc.at[pl.ds(s * _RPT + r * _C, _C)])
    rem = _RPT - nfull * _C
    if rem:
        pltpu.sync_copy(rows0.at[pl.ds(0, rem)],
                        acc.at[pl.ds(s * _RPT + nfull * _C, rem)])
    plsc.subcore_barrier()

    base = (c * 16 + s) * _EPT

    def load_idx(t, bi):
        off = base + t * _C
        pltpu.sync_copy(rowh.at[pl.ds(off, _C)], rowv[bi])
        pltpu.sync_copy(colh.at[pl.ds(off, _C)], colv[bi])
        for j in range(_C // 16):
            pos = t * _C + j * 16 + lane
            valid = pos < _EPT
            rv = rowv[bi][pl.ds(j * 16, 16)]
            cv = colv[bi][pl.ds(j * 16, 16)]
            rowv[bi][pl.ds(j * 16, 16)] = jnp.where(valid, rv, _N)
            colv[bi][pl.ds(j * 16, 16)] = jnp.where(valid, cv, 0)

    def start_gather(bi):
        pltpu.async_copy(vals.at[colv[bi]], rows[bi], semg[bi])

    def wait_gather(bi):
        pltpu.make_async_copy(vals.at[colv[bi]], rows[bi], semg[bi]).wait()

    def start_scatter(bi):
        pltpu.async_copy(rows[bi], acc.at[rowv[bi]], sems[bi], add=True)

    def wait_scatter(bi):
        pltpu.make_async_copy(rows[bi], acc.at[rowv[bi]], sems[bi]).wait()

    load_idx(0, 0)
    start_gather(0)
    for j in range(_C // 16):
        rowv[1][pl.ds(j * 16, 16)] = jnp.zeros((16,), jnp.int32) + _N
    start_scatter(1)

    def _pair(p, _):
        t1 = 2 * p + 1
        t2 = 2 * p + 2
        wait_gather(0)
        wait_scatter(1)
        load_idx(t1, 1)
        start_gather(1)
        start_scatter(0)
        wait_gather(1)
        wait_scatter(0)
        load_idx(t2, 0)
        start_gather(0)
        start_scatter(1)
        return 0
    assert _NCHUNK == 79
    lax.fori_loop(0, (_NCHUNK - 1) // 2, _pair, 0)
    wait_gather(0)
    wait_scatter(1)
    start_scatter(0)
    wait_scatter(0)

    plsc.subcore_barrier()
    pltpu.sync_copy(acc.at[pl.ds(s * _RPT, _RPT)],
                    out.at[c, pl.ds(s * _RPT, _RPT)])


@functools.lru_cache(maxsize=None)
def _edgepass_fn(d):
    return pl.kernel(
        _edgepass_body,
        out_type=jax.ShapeDtypeStruct((2, _NPAD, d), jnp.float32),
        mesh=_mesh(),
        scratch_types=[
            pltpu.VMEM((_C,), jnp.int32),
            pltpu.VMEM((_C,), jnp.int32),
            pltpu.VMEM((_C, d), jnp.float32),
            pltpu.VMEM((_C,), jnp.int32),
            pltpu.VMEM((_C,), jnp.int32),
            pltpu.VMEM((_C, d), jnp.float32),
            pltpu.VMEM_SHARED((_NPAD, d), jnp.float32),
            pltpu.SemaphoreType.DMA,
            pltpu.SemaphoreType.DMA,
            pltpu.SemaphoreType.DMA,
            pltpu.SemaphoreType.DMA,
        ],
    )


def _edgepass(vals, rowp, colp):
    return _edgepass_fn(vals.shape[1])(vals, rowp, colp)


_BLK = 1000


def _scales(cvec, mvec):
    degs = mvec * cvec + mvec
    a = mvec * jnp.where(degs > 0, lax.rsqrt(jnp.maximum(degs, 1e-30)), 0.0)
    degi = mvec * cvec
    av = mvec * jnp.where(degi > 0, lax.rsqrt(jnp.maximum(degi, 1e-30)), 0.0)
    return a, av


def _mm_body(x_ref, w_ref, c_ref, m_ref, xw_ref, v_ref):
    xw = jnp.dot(x_ref[...], w_ref[...], preferred_element_type=jnp.float32)
    a, _ = _scales(c_ref[...], m_ref[...])
    xw_ref[...] = xw
    v_ref[...] = xw * a


def _mm(x, W, cvec, mvec):
    n = x.shape[0]
    grid = n // _BLK
    return pl.pallas_call(
        _mm_body,
        grid=(grid,),
        in_specs=[
            pl.BlockSpec((_BLK, _D), lambda i: (i, 0)),
            pl.BlockSpec((_D, _D), lambda i: (0, 0)),
            pl.BlockSpec((_BLK, 1), lambda i: (i, 0)),
            pl.BlockSpec((_BLK, 1), lambda i: (i, 0)),
        ],
        out_specs=(
            pl.BlockSpec((_BLK, _D), lambda i: (i, 0)),
            pl.BlockSpec((_BLK, _D), lambda i: (i, 0)),
        ),
        out_shape=(
            jax.ShapeDtypeStruct((n, _D), jnp.float32),
            jax.ShapeDtypeStruct((n, _D), jnp.float32),
        ),
    )(x, W, cvec, mvec)


def _h_body(p0_ref, p1_ref, xw_ref, c_ref, m_ref, b_ref, h_ref, hv_ref):
    a, av = _scales(c_ref[...], m_ref[...])
    h = jax.nn.relu((p0_ref[...] + p1_ref[...]) * a
                    + xw_ref[...] * a * a + b_ref[...])
    h_ref[...] = h
    hv_ref[...] = h * av


def _h_stage(p0, p1, xw, cvec, mvec, b):
    n = xw.shape[0]
    grid = n // _BLK
    return pl.pallas_call(
        _h_body,
        grid=(grid,),
        in_specs=[
            pl.BlockSpec((_BLK, _D), lambda i: (i, 0)),
            pl.BlockSpec((_BLK, _D), lambda i: (i, 0)),
            pl.BlockSpec((_BLK, _D), lambda i: (i, 0)),
            pl.BlockSpec((_BLK, 1), lambda i: (i, 0)),
            pl.BlockSpec((_BLK, 1), lambda i: (i, 0)),
            pl.BlockSpec((1, _D), lambda i: (0, 0)),
        ],
        out_specs=(
            pl.BlockSpec((_BLK, _D), lambda i: (i, 0)),
            pl.BlockSpec((_BLK, _D), lambda i: (i, 0)),
        ),
        out_shape=(
            jax.ShapeDtypeStruct((n, _D), jnp.float32),
            jax.ShapeDtypeStruct((n, _D), jnp.float32),
        ),
    )(p0, p1, xw, cvec, mvec, b)


def _s_body(h_ref, q0_ref, q1_ref, c_ref, m_ref, s_ref):
    _, av = _scales(c_ref[...], m_ref[...])
    s_ref[...] = jnp.sum(
        jnp.abs(h_ref[...] - (q0_ref[...] + q1_ref[...]) * av),
        axis=1, keepdims=True)


def _s_stage(h, q0, q1, cvec, mvec):
    n = h.shape[0]
    grid = n // _BLK
    return pl.pallas_call(
        _s_body,
        grid=(grid,),
        in_specs=[
            pl.BlockSpec((_BLK, _D), lambda i: (i, 0)),
            pl.BlockSpec((_BLK, _D), lambda i: (i, 0)),
            pl.BlockSpec((_BLK, _D), lambda i: (i, 0)),
            pl.BlockSpec((_BLK, 1), lambda i: (i, 0)),
            pl.BlockSpec((_BLK, 1), lambda i: (i, 0)),
        ],
        out_specs=pl.BlockSpec((_BLK, 1), lambda i: (i, 0)),
        out_shape=jax.ShapeDtypeStruct((n, 1), jnp.float32),
    )(h, q0, q1, cvec, mvec)


def _out_body(x1m, x1s, c1, x2m, x2s, x3m, x3s, c2, w_ref, b_ref, o_ref):
    def xcat(xm, xs, cc):
        return jnp.concatenate(
            [jax.nn.relu(xm[...]),
             jax.nn.relu(xs[...] / jnp.maximum(cc[...], 1.0))], axis=1)
    xc = (xcat(x1m, x1s, c1) + xcat(x2m, x2s, c2) + xcat(x3m, x3s, c2))
    o_ref[...] = jnp.dot(xc, w_ref[...],
                         preferred_element_type=jnp.float32) + b_ref[...]


def _out_stage(x1m, x1s, c1, x2m, x2s, x3m, x3s, c2, linW, linb):
    return pl.pallas_call(
        _out_body,
        out_shape=jax.ShapeDtypeStruct((_G, linW.shape[1]), jnp.float32),
    )(x1m, x1s, c1, x2m, x2s, x3m, x3s, c2, linW, linb.reshape(1, -1))


def _topk_m(score, batch, nmask, g):
    n = score.shape[0]
    cnt_all = jnp.zeros((g,), jnp.int32).at[batch].add(1)
    cnt_valid = jnp.zeros((g,), jnp.int32).at[batch].add(
        nmask.astype(jnp.int32))
    k = jnp.ceil(0.5 * cnt_valid.astype(jnp.float32)).astype(jnp.int32)
    key = jnp.where(nmask, -score, jnp.inf)
    order = jnp.lexsort((key, batch))
    starts = jnp.cumsum(cnt_all) - cnt_all
    bo = batch[order]
    rank = jnp.arange(n, dtype=jnp.int32) - starts[bo]
    sel = (rank < k[bo]) & nmask[order]
    m = jnp.zeros((n,), bool).at[order].set(sel)
    return m, k


def _pool(h, m, batch, g):
    hm = jnp.where(m[:, None], h, 0.0)
    xmax = jax.ops.segment_max(hm, batch, num_segments=g)
    xsum = jax.ops.segment_sum(hm, batch, num_segments=g)
    return xmax, xsum


def kernel(x, edge_index, batch, W1, b1, W2, b2, W3, b3, linW, linb):
    n = x.shape[0]
    g = _G
    f32 = x.dtype
    row = edge_index[1]
    col = edge_index[0]
    epad = _NCHUNK * _C * 32 - _E
    rowp = jnp.concatenate([row, jnp.zeros((epad,), row.dtype)])
    colp = jnp.concatenate([col, jnp.zeros((epad,), col.dtype)])
    ones_col = jnp.ones((n, 1), f32)

    def agg(v):
        p = _edgepass(v, rowp, colp)
        return p[0, :n], p[1, :n]

    i0, i1 = agg(jnp.ones((n, _D), f32))
    indeg = (i0[:, :1] + i1[:, :1])

    xw1, v1 = _mm(x, W1, indeg, ones_col)
    p0, p1 = agg(v1)
    h1, hv1 = _h_stage(p0, p1, xw1, indeg, ones_col, b1.reshape(1, -1))
    q0, q1 = agg(hv1)
    s1 = _s_stage(h1, q0, q1, indeg, ones_col)[:, 0]

    m1, k1 = _topk_m(s1, batch, jnp.ones((n,), bool), g)
    m1p = m1.astype(f32)
    cnt1 = k1[:, None].astype(f32)
    x1m, x1s = _pool(h1, m1, batch, g)

    c0, c1 = agg(jnp.broadcast_to(m1p[:, None], (n, _D)))
    c2v = c0[:, :1] + c1[:, :1]
    m1c = m1p[:, None]

    xw2, v2 = _mm(h1, W2, c2v, m1c)
    p0, p1 = agg(v2)
    h2, hv2 = _h_stage(p0, p1, xw2, c2v, m1c, b2.reshape(1, -1))
    q0, q1 = agg(hv2)
    s2 = _s_stage(h2, q0, q1, c2v, m1c)[:, 0]

    m2, k2 = _topk_m(s2, batch, m1, g)
    m2p = m2.astype(f32)
    cnt2 = k2[:, None].astype(f32)
    x2m, x2s = _pool(h2, m2, batch, g)

    d0, d1 = agg(jnp.broadcast_to(m2p[:, None], (n, _D)))
    c3v = d0[:, :1] + d1[:, :1]
    m2c = m2p[:, None]

    xw3, v3 = _mm(h2, W3, c3v, m2c)
    p0, p1 = agg(v3)
    h3, _ = _h_stage(p0, p1, xw3, c3v, m2c, b3.reshape(1, -1))
    x3m, x3s = _pool(h3, m2, batch, g)

    return _out_stage(x1m, x1s, cnt1, x2m, x2s, x3m, x3s, cnt2,
                      linW, linb)
